# baseline (device time: 43682 ns/iter reference)
import functools

import jax
import jax.numpy as jnp
from jax import lax
from jax.experimental import pallas as pl
from jax.experimental.pallas import tpu as pltpu

N_DEV = 32
N_LAYERS = 3

GROUPS = (
    ((1, 0, 0), (0, 1, 0), (1, 1, 0)),
    ((0, 2, 0), (0, 0, 2), (0, 2, 2)),
    ((0, 0, 1),),
)
N_XFERS = sum(len(g) for g in GROUPS)
N_GROUPS = len(GROUPS)


def _partner(i, mask):
    mx, my, mz = mask
    z = i // 8
    p = i % 8
    y = p // 2
    q = p % 2
    x = jnp.where(y % 2 == 0, q, 1 - q)
    x = x ^ mx
    y = y ^ my
    z = z ^ mz
    q = jnp.where(y % 2 == 0, x, 1 - x)
    return z * 8 + y * 2 + q


def kernel(x, Win0, Wout0, Win1, Wout1, Win2, Wout2):
    b, d = x.shape

    def body(
        x_ref,
        win0_ref,
        wout0_ref,
        win1_ref,
        wout1_ref,
        win2_ref,
        wout2_ref,
        out_ref,
        send_buf,
        recv_buf,
        send_sems,
        recv_sems,
    ):
        my = lax.axis_index("i")
        partners = [[_partner(my, m) for m in grp] for grp in GROUPS]
        flat_partners = [p for grp in partners for p in grp]

        barrier = pltpu.get_barrier_semaphore()
        for ptn in flat_partners:
            pl.semaphore_signal(
                barrier,
                inc=1,
                device_id=(ptn,),
                device_id_type=pl.DeviceIdType.MESH,
            )

        win_refs = [win0_ref, win1_ref, win2_ref]
        wout_refs = [wout0_ref, wout1_ref, wout2_ref]

        pending_sends = []
        xv = x_ref[...].astype(jnp.bfloat16)
        for layer in range(N_LAYERS):
            h = jnp.maximum(
                jnp.dot(
                    xv,
                    win_refs[layer][...].astype(jnp.bfloat16),
                    preferred_element_type=jnp.float32,
                ),
                0.0,
            ).astype(jnp.bfloat16)
            acc = jnp.dot(
                h,
                wout_refs[layer][...].astype(jnp.bfloat16),
                preferred_element_type=jnp.float32,
            )
            xfer = layer * N_XFERS
            for g, grp in enumerate(GROUPS):
                sslot = layer * N_GROUPS + g
                send_buf[sslot, :, :] = acc.astype(jnp.bfloat16)
                if layer == 0 and g == 0:
                    pl.semaphore_wait(barrier, N_XFERS)
                rdmas = []
                for j in range(len(grp)):
                    rdma = pltpu.make_async_remote_copy(
                        src_ref=send_buf.at[sslot],
                        dst_ref=recv_buf.at[xfer],
                        send_sem=send_sems.at[xfer],
                        recv_sem=recv_sems.at[xfer],
                        device_id=(partners[g][j],),
                        device_id_type=pl.DeviceIdType.MESH,
                    )
                    rdma.start()
                    pending_sends.append(rdma)
                    rdmas.append((rdma, xfer))
                    xfer += 1
                for rdma, k in rdmas:
                    rdma.wait_recv()
                    acc = acc + recv_buf[k, :, :].astype(jnp.float32)
            xv = acc.astype(jnp.bfloat16)

        out_ref[...] = acc

        for rdma in pending_sends:
            rdma.wait_send()

        @functools.partial(pl.run_scoped, sem=pltpu.SemaphoreType.REGULAR)
        def _(sem):
            for ptn in flat_partners:
                pl.semaphore_signal(
                    sem,
                    inc=1,
                    device_id=(ptn,),
                    device_id_type=pl.DeviceIdType.MESH,
                )
            pl.semaphore_wait(sem, N_XFERS)

    total_slots = N_LAYERS * N_XFERS
    return pl.pallas_call(
        body,
        out_shape=jax.ShapeDtypeStruct((b, d), jnp.float32),
        in_specs=[pl.BlockSpec(memory_space=pltpu.VMEM)] * 7,
        out_specs=pl.BlockSpec(memory_space=pltpu.VMEM),
        scratch_shapes=[
            pltpu.VMEM((N_LAYERS * N_GROUPS, b, d), jnp.bfloat16),
            pltpu.VMEM((total_slots, b, d), jnp.bfloat16),
            pltpu.SemaphoreType.DMA((total_slots,)),
            pltpu.SemaphoreType.DMA((total_slots,)),
        ],
        compiler_params=pltpu.CompilerParams(collective_id=0),
    )(x, Win0, Wout0, Win1, Wout1, Win2, Wout2)


# device time: 41067 ns/iter; 1.0637x vs baseline; 1.0637x over previous
import functools

import jax
import jax.numpy as jnp
from jax import lax
from jax.experimental import pallas as pl
from jax.experimental.pallas import tpu as pltpu

N_DEV = 32
N_LAYERS = 3

GROUPS = (
    (
        (1, 0, 0),
        (0, 1, 0),
        (0, 0, 1),
        (1, 1, 0),
        (1, 0, 1),
        (0, 1, 1),
        (1, 1, 1),
    ),
    ((0, 2, 0), (0, 0, 2), (0, 2, 2)),
)
N_XFERS = sum(len(g) for g in GROUPS)
N_GROUPS = len(GROUPS)


def _partner(i, mask):
    mx, my, mz = mask
    z = i // 8
    p = i % 8
    y = p // 2
    q = p % 2
    x = jnp.where(y % 2 == 0, q, 1 - q)
    x = x ^ mx
    y = y ^ my
    z = z ^ mz
    q = jnp.where(y % 2 == 0, x, 1 - x)
    return z * 8 + y * 2 + q


def kernel(x, Win0, Wout0, Win1, Wout1, Win2, Wout2):
    b, d = x.shape

    def body(
        x_ref,
        win0_ref,
        wout0_ref,
        win1_ref,
        wout1_ref,
        win2_ref,
        wout2_ref,
        out_ref,
        send_buf,
        recv_buf,
        send_sems,
        recv_sems,
    ):
        my = lax.axis_index("i")
        partners = [[_partner(my, m) for m in grp] for grp in GROUPS]
        flat_partners = [p for grp in partners for p in grp]

        barrier = pltpu.get_barrier_semaphore()
        for ptn in flat_partners:
            pl.semaphore_signal(
                barrier,
                inc=1,
                device_id=(ptn,),
                device_id_type=pl.DeviceIdType.MESH,
            )

        win_refs = [win0_ref, win1_ref, win2_ref]
        wout_refs = [wout0_ref, wout1_ref, wout2_ref]

        pending_sends = []
        xv = x_ref[...].astype(jnp.bfloat16)
        for layer in range(N_LAYERS):
            h = jnp.maximum(
                jnp.dot(
                    xv,
                    win_refs[layer][...].astype(jnp.bfloat16),
                    preferred_element_type=jnp.float32,
                ),
                0.0,
            ).astype(jnp.bfloat16)
            acc = jnp.dot(
                h,
                wout_refs[layer][...].astype(jnp.bfloat16),
                preferred_element_type=jnp.float32,
            )
            xfer = layer * N_XFERS
            for g, grp in enumerate(GROUPS):
                sslot = layer * N_GROUPS + g
                send_buf[sslot, :, :] = acc.astype(jnp.bfloat16)
                if layer == 0 and g == 0:
                    pl.semaphore_wait(barrier, N_XFERS)
                rdmas = []
                for j in range(len(grp)):
                    rdma = pltpu.make_async_remote_copy(
                        src_ref=send_buf.at[sslot],
                        dst_ref=recv_buf.at[xfer],
                        send_sem=send_sems.at[xfer],
                        recv_sem=recv_sems.at[xfer],
                        device_id=(partners[g][j],),
                        device_id_type=pl.DeviceIdType.MESH,
                    )
                    rdma.start()
                    pending_sends.append(rdma)
                    rdmas.append((rdma, xfer))
                    xfer += 1
                for rdma, k in rdmas:
                    rdma.wait_recv()
                    acc = acc + recv_buf[k, :, :].astype(jnp.float32)
            xv = acc.astype(jnp.bfloat16)

        out_ref[...] = acc

        for rdma in pending_sends:
            rdma.wait_send()

        @functools.partial(pl.run_scoped, sem=pltpu.SemaphoreType.REGULAR)
        def _(sem):
            for ptn in flat_partners:
                pl.semaphore_signal(
                    sem,
                    inc=1,
                    device_id=(ptn,),
                    device_id_type=pl.DeviceIdType.MESH,
                )
            pl.semaphore_wait(sem, N_XFERS)

    total_slots = N_LAYERS * N_XFERS
    return pl.pallas_call(
        body,
        out_shape=jax.ShapeDtypeStruct((b, d), jnp.float32),
        in_specs=[pl.BlockSpec(memory_space=pltpu.VMEM)] * 7,
        out_specs=pl.BlockSpec(memory_space=pltpu.VMEM),
        scratch_shapes=[
            pltpu.VMEM((N_LAYERS * N_GROUPS, b, d), jnp.bfloat16),
            pltpu.VMEM((total_slots, b, d), jnp.bfloat16),
            pltpu.SemaphoreType.DMA((total_slots,)),
            pltpu.SemaphoreType.DMA((total_slots,)),
        ],
        compiler_params=pltpu.CompilerParams(collective_id=0),
    )(x, Win0, Wout0, Win1, Wout1, Win2, Wout2)


# device time: 37525 ns/iter; 1.1641x vs baseline; 1.0944x over previous
import functools

import jax
import jax.numpy as jnp
from jax import lax
from jax.experimental import pallas as pl
from jax.experimental.pallas import tpu as pltpu

N_DEV = 32
N_LAYERS = 3

GROUPS = (
    (
        (1, 1, 1),
        (1, 1, 0),
        (1, 0, 1),
        (0, 1, 1),
        (1, 0, 0),
        (0, 1, 0),
        (0, 0, 1),
    ),
    ((0, 2, 2), (0, 2, 0), (0, 0, 2)),
)
N_XFERS = sum(len(g) for g in GROUPS)
N_GROUPS = len(GROUPS)


def _partner(i, mask):
    mx, my, mz = mask
    z = i // 8
    p = i % 8
    y = p // 2
    q = p % 2
    x = jnp.where(y % 2 == 0, q, 1 - q)
    x = x ^ mx
    y = y ^ my
    z = z ^ mz
    q = jnp.where(y % 2 == 0, x, 1 - x)
    return z * 8 + y * 2 + q


def kernel(x, Win0, Wout0, Win1, Wout1, Win2, Wout2):
    b, d = x.shape

    def body(
        x_ref,
        win0_ref,
        wout0_ref,
        win1_ref,
        wout1_ref,
        win2_ref,
        wout2_ref,
        out_ref,
        send_buf,
        recv_buf,
        send_sems,
        recv_sems,
    ):
        my = lax.axis_index("i")
        partners = [[_partner(my, m) for m in grp] for grp in GROUPS]
        flat_partners = [p for grp in partners for p in grp]

        barrier = pltpu.get_barrier_semaphore()
        for ptn in flat_partners:
            pl.semaphore_signal(
                barrier,
                inc=1,
                device_id=(ptn,),
                device_id_type=pl.DeviceIdType.MESH,
            )

        win_refs = [win0_ref, win1_ref, win2_ref]
        wout_refs = [wout0_ref, wout1_ref, wout2_ref]

        pending_sends = []
        xv = x_ref[...].astype(jnp.bfloat16)
        for layer in range(N_LAYERS):
            h = jnp.maximum(
                jnp.dot(
                    xv,
                    win_refs[layer][...].astype(jnp.bfloat16),
                    preferred_element_type=jnp.float32,
                ),
                0.0,
            ).astype(jnp.bfloat16)
            acc = jnp.dot(
                h,
                wout_refs[layer][...].astype(jnp.bfloat16),
                preferred_element_type=jnp.float32,
            )
            xfer = layer * N_XFERS
            for g, grp in enumerate(GROUPS):
                sslot = layer * N_GROUPS + g
                send_buf[sslot, :, :] = acc.astype(jnp.bfloat16)
                if layer == 0 and g == 0:
                    pl.semaphore_wait(barrier, N_XFERS)
                rdmas = []
                for j in range(len(grp)):
                    rdma = pltpu.make_async_remote_copy(
                        src_ref=send_buf.at[sslot],
                        dst_ref=recv_buf.at[xfer],
                        send_sem=send_sems.at[xfer],
                        recv_sem=recv_sems.at[xfer],
                        device_id=(partners[g][j],),
                        device_id_type=pl.DeviceIdType.MESH,
                    )
                    rdma.start()
                    pending_sends.append(rdma)
                    rdmas.append((rdma, xfer))
                    xfer += 1
                for rdma, k in rdmas:
                    rdma.wait_recv()
                    acc = acc + recv_buf[k, :, :].astype(jnp.float32)
            xv = acc.astype(jnp.bfloat16)

        out_ref[...] = acc

        for rdma in pending_sends:
            rdma.wait_send()

        @functools.partial(pl.run_scoped, sem=pltpu.SemaphoreType.REGULAR)
        def _(sem):
            for ptn in flat_partners:
                pl.semaphore_signal(
                    sem,
                    inc=1,
                    device_id=(ptn,),
                    device_id_type=pl.DeviceIdType.MESH,
                )
            pl.semaphore_wait(sem, N_XFERS)

    total_slots = N_LAYERS * N_XFERS
    return pl.pallas_call(
        body,
        out_shape=jax.ShapeDtypeStruct((b, d), jnp.float32),
        in_specs=[pl.BlockSpec(memory_space=pltpu.VMEM)] * 7,
        out_specs=pl.BlockSpec(memory_space=pltpu.VMEM),
        scratch_shapes=[
            pltpu.VMEM((N_LAYERS * N_GROUPS, b, d), jnp.bfloat16),
            pltpu.VMEM((total_slots, b, d), jnp.bfloat16),
            pltpu.SemaphoreType.DMA((total_slots,)),
            pltpu.SemaphoreType.DMA((total_slots,)),
        ],
        compiler_params=pltpu.CompilerParams(collective_id=0),
    )(x, Win0, Wout0, Win1, Wout1, Win2, Wout2)
